# Initial kernel scaffold; baseline (speedup 1.0000x reference)
import jax
import jax.numpy as jnp
from jax.experimental import pallas as pl

N = 100000
E = 1600000
G = 64
EPS = 1e-5


def _gatv2(x, ei, Wl, bl, Wr, br, att, bias):
    src = ei[0]
    dst = ei[1]
    xl = x @ Wl + bl
    xr = x @ Wr + br
    e = jax.nn.leaky_relu(xl[src] + xr[dst], 0.2)
    a = e @ att
    amax = jax.lax.stop_gradient(jax.ops.segment_max(a, dst, num_segments=N))
    amax = jnp.where(jnp.isfinite(amax), amax, 0.0)
    ex = jnp.exp(a - amax[dst])
    den = jax.ops.segment_sum(ex, dst, num_segments=N)
    w = ex / (den[dst] + 1e-16)
    out = jax.ops.segment_sum(xl[src] * w[:, None], dst, num_segments=N)
    return out + bias


def _graphnorm(x, batch, w, b, ms):
    cnt = jnp.maximum(jax.ops.segment_sum(jnp.ones((x.shape[0],), x.dtype), batch, num_segments=G), 1.0)
    mean = jax.ops.segment_sum(x, batch, num_segments=G) / cnt[:, None]
    out = x - ms * mean[batch]
    var = jax.ops.segment_sum(out * out, batch, num_segments=G) / cnt[:, None]
    return w * out / jnp.sqrt(var[batch] + EPS) + b


def _gmp(x, batch):
    cnt = jnp.maximum(jax.ops.segment_sum(jnp.ones((x.shape[0],), x.dtype), batch, num_segments=G), 1.0)
    return jax.ops.segment_sum(x, batch, num_segments=G) / cnt[:, None]


def _id_kernel(x_ref, o_ref):
    o_ref[...] = x_ref[...]


def kernel(x, edge_index, batch, Wl1, bl1, Wr1, br1, att1, bias1, gw1, gb1, gm1, Wl2, bl2, Wr2, br2, att2, bias2, gw2, gb2, gm2, Wl3, bl3, Wr3, br3, att3, bias3, gw3, gb3, gm3, Wlin, blin):
    h = _gatv2(x, edge_index, Wl1, bl1, Wr1, br1, att1, bias1)
    h = jax.nn.relu(h)
    h = _graphnorm(h, batch, gw1, gb1, gm1)
    h = _gatv2(h, edge_index, Wl2, bl2, Wr2, br2, att2, bias2)
    h = jax.nn.relu(h)
    h = _graphnorm(h, batch, gw2, gb2, gm2)
    h = _gatv2(h, edge_index, Wl3, bl3, Wr3, br3, att3, bias3)
    h = _graphnorm(h, batch, gw3, gb3, gm3)
    w3 = _gmp(h, batch)
    o = w3 @ Wlin + blin
    o = pl.pallas_call(_id_kernel, out_shape=jax.ShapeDtypeStruct(o.shape, o.dtype))(o)
    return (o, w3)


# speedup vs baseline: 3.8734x; 3.8734x over previous
import functools

import jax
import jax.numpy as jnp
from jax import lax
from jax.experimental import pallas as pl
from jax.experimental.pallas import tpu as pltpu
from jax.experimental.pallas import tpu_sc as plsc

N = 100000
E = 1600000
H = 32
G = 64
EPS = 1e-5

NPT = 3136
NPAD = 32 * NPT
C = 128
NGRP = C // 16
EPAD = E + C

_MESH = plsc.VectorSubcoreMesh(core_axis_name="c", subcore_axis_name="s")
_CP = pltpu.CompilerParams(needs_layout_passes=False, use_tc_tiling_on_sc=False)


def _wid():
    return lax.axis_index("s") * 2 + lax.axis_index("c")


def _bound_at(bv, w):
    total = jnp.int32(0)
    for k in range(3):
        v = bv[pl.ds(16 * k, 16)]
        lane = lax.iota(jnp.int32, 16)
        total = total + jnp.sum(jnp.where(lane == (w - 16 * k), v, 0))
    return total


def _edge_range(bounds_ref, bv, w):
    pltpu.sync_copy(bounds_ref, bv)
    e_start = _bound_at(bv, w)
    e_end = _bound_at(bv, w + 1)
    base0 = (e_start // 8) * 8
    nchunks = (e_end - base0 + (C - 1)) // C
    return e_start, e_end, base0, nchunks


@functools.partial(
    pl.kernel, mesh=_MESH, compiler_params=_CP,
    out_type=jax.ShapeDtypeStruct((EPAD, H), jnp.float32),
    scratch_types=[
        pltpu.VMEM((48,), jnp.int32),
        pltpu.VMEM((C,), jnp.int32),
        pltpu.VMEM((C,), jnp.int32),
        pltpu.VMEM((C, H), jnp.float32),
        pltpu.VMEM((C, H), jnp.float32),
        pltpu.VMEM((C, H), jnp.float32),
        pltpu.SemaphoreType.DMA,
        pltpu.SemaphoreType.DMA,
    ],
)
def _attention_inputs(xl_hbm, xr_hbm, srcs_hbm, dsts_hbm, bounds_hbm,
                      e_hbm, bv, sv, dv, xlr, xrr, ev, sem1, sem2):
    w = _wid()
    e_start, e_end, base0, nchunks = _edge_range(bounds_hbm, bv, w)

    def chunk(k, carry):
        base = base0 + k * C
        pltpu.sync_copy(srcs_hbm.at[pl.ds(base, C)], sv)
        pltpu.sync_copy(dsts_hbm.at[pl.ds(base, C)], dv)
        cp1 = pltpu.async_copy(xl_hbm.at[sv], xlr, sem1)
        cp2 = pltpu.async_copy(xr_hbm.at[dv], xrr, sem2)
        cp1.wait()
        cp2.wait()

        def row(r, carry2):
            for half in (0, 16):
                t = xlr[r, pl.ds(half, 16)] + xrr[r, pl.ds(half, 16)]
                ev[r, pl.ds(half, 16)] = jnp.where(t >= 0, t, jnp.float32(0.2) * t)
            return carry2

        lax.fori_loop(0, C, row, 0)
        pltpu.sync_copy(ev, e_hbm.at[pl.ds(base, C)])
        return carry

    lax.fori_loop(0, nchunks, chunk, 0)


@functools.partial(
    pl.kernel, mesh=_MESH, compiler_params=_CP,
    out_type=jax.ShapeDtypeStruct((NPAD, H), jnp.float32),
    scratch_types=[
        pltpu.VMEM((48,), jnp.int32),
        pltpu.VMEM((NPT,), jnp.float32),
        pltpu.VMEM((NPT,), jnp.float32),
        pltpu.VMEM((NPT, H), jnp.float32),
        pltpu.VMEM((C,), jnp.int32),
        pltpu.VMEM((C,), jnp.int32),
        pltpu.VMEM((C,), jnp.float32),
        pltpu.VMEM((C, H), jnp.float32),
        pltpu.SemaphoreType.DMA,
    ],
)
def _aggregate(xl_hbm, a_hbm, amax_hbm, srcs_hbm, dsts_hbm, bounds_hbm,
               num_hbm, bv, amv, denv, accv, sv, dv, av, xlr, sem1):
    w = _wid()
    e_start, e_end, base0, nchunks = _edge_range(bounds_hbm, bv, w)
    n0 = w * NPT
    pltpu.sync_copy(amax_hbm.at[pl.ds(n0, NPT)], amv)
    lane = lax.iota(jnp.int32, 16)
    zero16 = jnp.zeros((16,), jnp.float32)

    def zrow(r, carry):
        accv[r, pl.ds(0, 16)] = zero16
        accv[r, pl.ds(16, 16)] = zero16
        return carry

    lax.fori_loop(0, NPT, zrow, 0)

    def zden(i, carry):
        denv[pl.ds(i * 16, 16)] = zero16
        return carry

    lax.fori_loop(0, NPT // 16, zden, 0)

    def group_ex(base, g):
        eid = base + g * 16 + lane
        d = dv[pl.ds(g * 16, 16)]
        a = av[pl.ds(g * 16, 16)]
        valid = (eid >= e_start) & (eid < e_end)
        dl = d - n0
        dl = jnp.where(valid, dl, 0)
        am = plsc.load_gather(amv, [dl])
        ex = jnp.exp(a - am)
        ex = jnp.where(valid, ex, jnp.float32(0.0))
        return dl, ex

    def chunk_a(k, carry):
        base = base0 + k * C
        pltpu.sync_copy(dsts_hbm.at[pl.ds(base, C)], dv)
        pltpu.sync_copy(a_hbm.at[pl.ds(base, C)], av)
        for g in range(NGRP):
            dl, ex = group_ex(base, g)
            plsc.addupdate_scatter(denv, [dl], ex)
        return carry

    lax.fori_loop(0, nchunks, chunk_a, 0)

    def chunk_b(k, carry):
        base = base0 + k * C
        pltpu.sync_copy(srcs_hbm.at[pl.ds(base, C)], sv)
        pltpu.sync_copy(dsts_hbm.at[pl.ds(base, C)], dv)
        pltpu.sync_copy(a_hbm.at[pl.ds(base, C)], av)
        pltpu.async_copy(xl_hbm.at[sv], xlr, sem1).wait()
        for g in range(NGRP):
            dl, ex = group_ex(base, g)
            den = plsc.load_gather(denv, [dl])
            wv = ex / (den + jnp.float32(1e-16))
            rows = lane + g * 16
            for h in range(H):
                col = jnp.full((16,), h, jnp.int32)
                xlh = plsc.load_gather(xlr, [rows, col])
                m = xlh * wv
                plsc.addupdate_scatter(accv, [dl, col], m)
        return carry

    lax.fori_loop(0, nchunks, chunk_b, 0)
    pltpu.sync_copy(accv, num_hbm.at[pl.ds(n0, NPT)])


def _graphnorm(x, batch, w, b, ms):
    cnt = jnp.maximum(jax.ops.segment_sum(jnp.ones((x.shape[0],), x.dtype), batch, num_segments=G), 1.0)
    mean = jax.ops.segment_sum(x, batch, num_segments=G) / cnt[:, None]
    out = x - ms * mean[batch]
    var = jax.ops.segment_sum(out * out, batch, num_segments=G) / cnt[:, None]
    return w * out / jnp.sqrt(var[batch] + EPS) + b


def _gmp(x, batch):
    cnt = jnp.maximum(jax.ops.segment_sum(jnp.ones((x.shape[0],), x.dtype), batch, num_segments=G), 1.0)
    return jax.ops.segment_sum(x, batch, num_segments=G) / cnt[:, None]


def _gat_layer(h, srcs, dsts, bounds, Wl, bl, Wr, br, att, bias):
    xl = h @ Wl + bl
    xr = h @ Wr + br
    e_pad = _attention_inputs(xl, xr, srcs, dsts, bounds)
    a_pad = e_pad @ att
    a = a_pad[:E]
    amax = lax.stop_gradient(jax.ops.segment_max(a, dsts[:E], num_segments=N))
    amax = jnp.where(jnp.isfinite(amax), amax, 0.0)
    amax_pad = jnp.pad(amax, (0, NPAD - N))
    num = _aggregate(xl, a_pad, amax_pad, srcs, dsts, bounds)
    return num[:N] + bias


def kernel(x, edge_index, batch, Wl1, bl1, Wr1, br1, att1, bias1, gw1, gb1, gm1, Wl2, bl2, Wr2, br2, att2, bias2, gw2, gb2, gm2, Wl3, bl3, Wr3, br3, att3, bias3, gw3, gb3, gm3, Wlin, blin):
    src, dst = edge_index[0], edge_index[1]
    order = jnp.argsort(dst, stable=True)
    dsts_s = dst[order]
    srcs_s = src[order]
    srcs = jnp.concatenate([srcs_s, jnp.zeros((EPAD - E,), jnp.int32)])
    dsts = jnp.concatenate([dsts_s, jnp.full((EPAD - E,), N - 1, jnp.int32)])
    bounds = jnp.searchsorted(dsts_s, jnp.arange(33, dtype=jnp.int32) * NPT, side="left").astype(jnp.int32)
    bounds = jnp.pad(bounds, (0, 48 - 33))

    h = _gat_layer(x, srcs, dsts, bounds, Wl1, bl1, Wr1, br1, att1, bias1)
    h = jax.nn.relu(h)
    h = _graphnorm(h, batch, gw1, gb1, gm1)
    h = _gat_layer(h, srcs, dsts, bounds, Wl2, bl2, Wr2, br2, att2, bias2)
    h = jax.nn.relu(h)
    h = _graphnorm(h, batch, gw2, gb2, gm2)
    h = _gat_layer(h, srcs, dsts, bounds, Wl3, bl3, Wr3, br3, att3, bias3)
    h = _graphnorm(h, batch, gw3, gb3, gm3)
    w3 = _gmp(h, batch)
    o = w3 @ Wlin + blin
    return (o, w3)
